# baseline (speedup 1.0000x reference)
import jax
import jax.numpy as jnp
from jax import lax
from jax.experimental import pallas as pl
from jax.experimental.pallas import tpu as pltpu
from jax.experimental.pallas import tpu_sc as plsc

_ROWS = 1536
_COLS = 512
_B_PER_BLOCK = 2
_NC = 2
_NTAB = 1000


def _sc_gather_body(t_hbm, sac_hbm, somac_hbm, a_hbm, s_hbm,
                    t_v, sac_v, somac_v, a_v, s_v):
    wid = lax.axis_index("s") * _NC + lax.axis_index("c")

    @pl.when(wid < 4)
    def _():
        base = wid * 16
        pltpu.sync_copy(t_hbm.at[pl.ds(base, 16)], t_v)
        pltpu.sync_copy(sac_hbm, sac_v.at[pl.ds(0, _NTAB)])
        pltpu.sync_copy(somac_hbm, somac_v.at[pl.ds(0, _NTAB)])
        idx = t_v[...]
        a_v[...] = plsc.load_gather(sac_v, [idx])
        s_v[...] = plsc.load_gather(somac_v, [idx])
        pltpu.sync_copy(a_v, a_hbm.at[pl.ds(base, 16)])
        pltpu.sync_copy(s_v, s_hbm.at[pl.ds(base, 16)])


def _sc_gather(t32, sac, somac):
    B = t32.shape[0]
    mesh = plsc.VectorSubcoreMesh(core_axis_name="c", subcore_axis_name="s")
    f = pl.kernel(
        _sc_gather_body,
        out_type=(
            jax.ShapeDtypeStruct((B,), jnp.float32),
            jax.ShapeDtypeStruct((B,), jnp.float32),
        ),
        mesh=mesh,
        compiler_params=pltpu.CompilerParams(needs_layout_passes=False),
        scratch_types=[
            pltpu.VMEM((16,), jnp.int32),
            pltpu.VMEM((1024,), jnp.float32),
            pltpu.VMEM((1024,), jnp.float32),
            pltpu.VMEM((16,), jnp.float32),
            pltpu.VMEM((16,), jnp.float32),
        ],
    )
    return f(t32, sac, somac)


def _qsample_body(a_ref, s_ref, x_ref, n_ref, o_ref):
    bb = pl.program_id(0)
    for k in range(_B_PER_BLOCK):
        b = bb * _B_PER_BLOCK + k
        o_ref[k] = a_ref[b] * x_ref[k] + s_ref[b] * n_ref[k]


def kernel(x_start, t, noise, sqrt_alphas_cumprod, sqrt_one_minus_alphas_cumprod):
    B, C, H, W = x_start.shape
    xr = x_start.reshape(B, _ROWS, _COLS)
    nr = noise.reshape(B, _ROWS, _COLS)
    t32 = t.astype(jnp.int32)

    a_vec, s_vec = _sc_gather(t32, sqrt_alphas_cumprod, sqrt_one_minus_alphas_cumprod)

    grid = (B // _B_PER_BLOCK,)
    spec = pl.BlockSpec((_B_PER_BLOCK, _ROWS, _COLS), lambda b, *_: (b, 0, 0))
    grid_spec = pltpu.PrefetchScalarGridSpec(
        num_scalar_prefetch=2,
        grid=grid,
        in_specs=[spec, spec],
        out_specs=spec,
    )
    out = pl.pallas_call(
        _qsample_body,
        grid_spec=grid_spec,
        out_shape=jax.ShapeDtypeStruct((B, _ROWS, _COLS), jnp.float32),
    )(a_vec, s_vec, xr, nr)
    return out.reshape(B, C, H, W)


# speedup vs baseline: 1.0140x; 1.0140x over previous
import jax
import jax.numpy as jnp
from jax import lax
from jax.experimental import pallas as pl
from jax.experimental.pallas import tpu as pltpu
from jax.experimental.pallas import tpu_sc as plsc

_ROWS = 1536
_COLS = 512
_B_PER_BLOCK = 2
_NC = 2
_NTAB = 1000


def _sc_gather_body(t_hbm, sac_hbm, somac_hbm, a_hbm, s_hbm,
                    t_v, sac_v, somac_v, a_v, s_v):
    wid = lax.axis_index("s") * _NC + lax.axis_index("c")

    @pl.when(wid < 4)
    def _():
        base = wid * 16
        pltpu.sync_copy(t_hbm.at[pl.ds(base, 16)], t_v)
        pltpu.sync_copy(sac_hbm, sac_v.at[pl.ds(0, _NTAB)])
        pltpu.sync_copy(somac_hbm, somac_v.at[pl.ds(0, _NTAB)])
        idx = t_v[...]
        a_v[...] = plsc.load_gather(sac_v, [idx])
        s_v[...] = plsc.load_gather(somac_v, [idx])
        pltpu.sync_copy(a_v, a_hbm.at[pl.ds(base, 16)])
        pltpu.sync_copy(s_v, s_hbm.at[pl.ds(base, 16)])


def _sc_gather(t32, sac, somac):
    B = t32.shape[0]
    mesh = plsc.VectorSubcoreMesh(core_axis_name="c", subcore_axis_name="s")
    f = pl.kernel(
        _sc_gather_body,
        out_type=(
            jax.ShapeDtypeStruct((B,), jnp.float32),
            jax.ShapeDtypeStruct((B,), jnp.float32),
        ),
        mesh=mesh,
        compiler_params=pltpu.CompilerParams(needs_layout_passes=False),
        scratch_types=[
            pltpu.VMEM((16,), jnp.int32),
            pltpu.VMEM((1024,), jnp.float32),
            pltpu.VMEM((1024,), jnp.float32),
            pltpu.VMEM((16,), jnp.float32),
            pltpu.VMEM((16,), jnp.float32),
        ],
    )
    return f(t32, sac, somac)


def _fma_lo_body(t_ref, sac_ref, somac_ref, x_ref, n_ref, o_ref):
    bb = pl.program_id(0)
    for k in range(_B_PER_BLOCK):
        tt = t_ref[bb * _B_PER_BLOCK + k]
        o_ref[k] = sac_ref[tt] * x_ref[k] + somac_ref[tt] * n_ref[k]


def _fma_hi_body(a_ref, s_ref, x_ref, n_ref, io_ref, o_ref):
    bb = pl.program_id(0)
    half = pl.num_programs(0) * _B_PER_BLOCK
    for k in range(_B_PER_BLOCK):
        b = half + bb * _B_PER_BLOCK + k
        o_ref[k] = a_ref[b] * x_ref[k] + s_ref[b] * n_ref[k]


def kernel(x_start, t, noise, sqrt_alphas_cumprod, sqrt_one_minus_alphas_cumprod):
    B, C, H, W = x_start.shape
    xr = x_start.reshape(B, _ROWS, _COLS)
    nr = noise.reshape(B, _ROWS, _COLS)
    t32 = t.astype(jnp.int32)
    half = B // 2

    a_vec, s_vec = _sc_gather(t32, sqrt_alphas_cumprod, sqrt_one_minus_alphas_cumprod)

    blk = (_B_PER_BLOCK, _ROWS, _COLS)
    grid = (half // _B_PER_BLOCK,)
    out_sds = jax.ShapeDtypeStruct((B, _ROWS, _COLS), jnp.float32)

    lo_spec = pl.BlockSpec(blk, lambda b, *_: (b, 0, 0))
    out1 = pl.pallas_call(
        _fma_lo_body,
        grid_spec=pltpu.PrefetchScalarGridSpec(
            num_scalar_prefetch=3,
            grid=grid,
            in_specs=[lo_spec, lo_spec],
            out_specs=lo_spec,
        ),
        out_shape=out_sds,
    )(t32, sqrt_alphas_cumprod, sqrt_one_minus_alphas_cumprod, xr, nr)

    nblk = half // _B_PER_BLOCK
    hi_spec = pl.BlockSpec(blk, lambda b, *_: (b + nblk, 0, 0))
    hbm_spec = pl.BlockSpec(memory_space=pltpu.MemorySpace.HBM)
    out = pl.pallas_call(
        _fma_hi_body,
        grid_spec=pltpu.PrefetchScalarGridSpec(
            num_scalar_prefetch=2,
            grid=grid,
            in_specs=[hi_spec, hi_spec, hbm_spec],
            out_specs=hi_spec,
        ),
        out_shape=out_sds,
        input_output_aliases={4: 0},
    )(a_vec, s_vec, xr, nr, out1)
    return out.reshape(B, C, H, W)


# speedup vs baseline: 1.1093x; 1.0939x over previous
import jax
import jax.numpy as jnp
from jax.experimental import pallas as pl
from jax.experimental.pallas import tpu as pltpu

_ROWS = 1536
_COLS = 512
_B_PER_BLOCK = 2


def _qsample_body(t_ref, sac_ref, somac_ref, x_ref, n_ref, o_ref):
    bb = pl.program_id(0)
    for k in range(_B_PER_BLOCK):
        tt = t_ref[bb * _B_PER_BLOCK + k]
        a = sac_ref[tt]
        s = somac_ref[tt]
        o_ref[k] = a * x_ref[k] + s * n_ref[k]


def kernel(x_start, t, noise, sqrt_alphas_cumprod, sqrt_one_minus_alphas_cumprod):
    B, C, H, W = x_start.shape
    xr = x_start.reshape(B, _ROWS, _COLS)
    nr = noise.reshape(B, _ROWS, _COLS)
    t32 = t.astype(jnp.int32)

    grid = (B // _B_PER_BLOCK,)
    spec = pl.BlockSpec((_B_PER_BLOCK, _ROWS, _COLS), lambda b, *_: (b, 0, 0))
    grid_spec = pltpu.PrefetchScalarGridSpec(
        num_scalar_prefetch=3,
        grid=grid,
        in_specs=[spec, spec],
        out_specs=spec,
    )
    out = pl.pallas_call(
        _qsample_body,
        grid_spec=grid_spec,
        out_shape=jax.ShapeDtypeStruct((B, _ROWS, _COLS), jnp.float32),
    )(t32, sqrt_alphas_cumprod, sqrt_one_minus_alphas_cumprod, xr, nr)
    return out.reshape(B, C, H, W)
